# baseline (speedup 1.0000x reference)
import functools

import jax
import jax.numpy as jnp
from jax import lax
from jax.experimental import pallas as pl
from jax.experimental.pallas import tpu as pltpu
from jax.experimental.pallas import tpu_sc as plsc

_N = 10000
_E = 320000
_D = 128

_NC = 2
_NS = 16
_NW = _NC * _NS
_EB = 40
_TPH = 8
_PH0 = 3
_PHB = 64
_EPP = _PHB * _EB
_EPW = _TPH * _EPP
_EPAD = _NS * _EPW
_NP = 10240
_RPT = _NP // _NS

_RN = 1000
_REB = 4000


def _node_body(x_ref, th_ref, We_ref, be_ref, FW1_ref, Fb1_ref, FW2_ref,
               Fb2_ref, GW1_ref, Gb1_ref, GW2_ref, Gb2_ref, Wo_ref, bo_ref,
               a_ref, b_ref, fo_ref, r_ref):
    h = x_ref[...] * We_ref[...] + be_ref[...]
    z = (jnp.dot(h, FW1_ref[0:_D, :], preferred_element_type=jnp.float32)
         + jnp.dot(th_ref[...], FW1_ref[_D:2 * _D, :],
                   preferred_element_type=jnp.float32)
         + Fb1_ref[...])
    u = jnp.where(z > 0, z, jnp.exp(z) - 1.0)
    w2o = jnp.dot(FW2_ref[...], Wo_ref[...], preferred_element_type=jnp.float32)
    fo_ref[...] = (jnp.dot(u, w2o, preferred_element_type=jnp.float32)
                   + jnp.dot(Fb2_ref[...], Wo_ref[...],
                             preferred_element_type=jnp.float32)
                   + bo_ref[...])
    a_ref[...] = (jnp.dot(h, GW1_ref[0:_D, :],
                          preferred_element_type=jnp.float32) + Gb1_ref[...])
    b_ref[...] = jnp.dot(h, GW1_ref[_D:2 * _D, :],
                         preferred_element_type=jnp.float32)
    g2o = jnp.dot(GW2_ref[...], Wo_ref[...], preferred_element_type=jnp.float32)
    gb2o = jnp.dot(Gb2_ref[...], Wo_ref[...], preferred_element_type=jnp.float32)
    r_ref[...] = (g2o * gb2o) / jnp.sum(g2o * g2o)


def _node_call(x2, theta, We, be2, FW1, Fb12, FW2, Fb22, GW1, Gb12, GW2, Gb22,
               Wo, bo2):
    full = lambda s: pl.BlockSpec(s, lambda i: (0, 0))
    return pl.pallas_call(
        _node_body,
        grid=(_N // _RN,),
        in_specs=[
            pl.BlockSpec((_RN, 1), lambda i: (i, 0)),
            pl.BlockSpec((_RN, _D), lambda i: (i, 0)),
            full((1, _D)), full((1, _D)),
            full((2 * _D, _D)), full((1, _D)),
            full((_D, _D)), full((1, _D)),
            full((3 * _D, _D)), full((1, _D)),
            full((_D, _D)), full((1, _D)),
            full((_D, 1)), full((1, 1)),
        ],
        out_specs=[
            pl.BlockSpec((_RN, _D), lambda i: (i, 0)),
            pl.BlockSpec((_RN, _D), lambda i: (i, 0)),
            pl.BlockSpec((_RN, 1), lambda i: (i, 0)),
            pl.BlockSpec((_D, 1), lambda i: (0, 0)),
        ],
        out_shape=[
            jax.ShapeDtypeStruct((_NP, _D), jnp.float32),
            jax.ShapeDtypeStruct((_NP, _D), jnp.float32),
            jax.ShapeDtypeStruct((_N, 1), jnp.float32),
            jax.ShapeDtypeStruct((_D, 1), jnp.float32),
        ],
    )(x2, theta, We, be2, FW1, Fb12, FW2, Fb22, GW1, Gb12, GW2, Gb22, Wo, bo2)


def _edge_body(w_ref, g_ref, c_ref):
    c_ref[...] = jnp.dot(w_ref[...], g_ref[...],
                         preferred_element_type=jnp.float32)


def _edge_call(w_edge, G1c):
    return pl.pallas_call(
        _edge_body,
        grid=(_E // _REB,),
        in_specs=[
            pl.BlockSpec((_REB, _D), lambda i: (i, 0)),
            pl.BlockSpec((_D, _D), lambda i: (0, 0)),
        ],
        out_specs=pl.BlockSpec((_REB, _D), lambda i: (i, 0)),
        out_shape=jax.ShapeDtypeStruct((_EPAD, _D), jnp.float32),
    )(w_edge, G1c)


def _sc_body(src_hbm, dst_hbm, a_hbm, b_hbm, c_hbm, r_hbm,
             s_out,
             idxs_v, idxd_v, a0_v, b0_v, c0_v, a1_v, b1_v, c1_v, r_v, s_sh,
             sa0, sb0, sc0, sa1, sb1, sc1):
    cid = lax.axis_index("c")
    sid = lax.axis_index("s")

    pltpu.sync_copy(r_hbm, r_v)

    zero16 = jnp.zeros((16,), jnp.float32)
    for q in range(_EB):
        for j in range(_D // 16):
            a0_v[q, pl.ds(j * 16, 16)] = zero16
    row0 = sid * _RPT
    for q in range(_RPT // _EB):
        pltpu.sync_copy(a0_v, s_sh.at[pl.ds(row0 + q * _EB, _EB)])
    plsc.subcore_barrier()

    nph = jnp.where(cid == 0, _PH0, _TPH - _PH0)
    phoff = jnp.where(cid == 0, 0, _PH0)

    def phase(ph, carry):
        phg = phoff + ph
        pltpu.sync_copy(src_hbm.at[sid, phg], idxs_v)
        pltpu.sync_copy(dst_hbm.at[sid, phg], idxd_v)
        base0 = sid * _EPW + phg * _EPP

        def issue(i, av, bv, cv, sa, sb, sc):
            pltpu.async_copy(a_hbm.at[idxd_v.at[i]], av, sa)
            pltpu.async_copy(b_hbm.at[idxs_v.at[i]], bv, sb)
            pltpu.async_copy(c_hbm.at[pl.ds(base0 + i * _EB, _EB)], cv, sc)

        def wait_set(av, bv, cv, sa, sb, sc):
            pltpu.make_async_copy(a_hbm.at[idxd_v.at[0]], av, sa).wait()
            pltpu.make_async_copy(b_hbm.at[idxs_v.at[0]], bv, sb).wait()
            pltpu.make_async_copy(c_hbm.at[pl.ds(base0, _EB)], cv, sc).wait()

        def process(i, av, bv, cv, sa, sb, sc):
            wait_set(av, bv, cv, sa, sb, sc)

            @plsc.parallel_loop(0, _EB, step=1, unroll=4)
            def comp(e):
                for j in range(_D // 16):
                    sl = pl.ds(j * 16, 16)
                    z = av[e, sl] + bv[e, sl] + cv[e, sl]
                    av[e, sl] = jnp.where(z > 0, z, jnp.exp(z) - 1.0) + r_v[sl]

            pltpu.sync_copy(av, s_sh.at[idxd_v.at[i]], add=True)

        issue(0, a0_v, b0_v, c0_v, sa0, sb0, sc0)

        def body(k, inner):
            i0 = 2 * k
            issue(i0 + 1, a1_v, b1_v, c1_v, sa1, sb1, sc1)
            process(i0, a0_v, b0_v, c0_v, sa0, sb0, sc0)
            issue(i0 + 2, a0_v, b0_v, c0_v, sa0, sb0, sc0)
            process(i0 + 1, a1_v, b1_v, c1_v, sa1, sb1, sc1)
            return inner

        lax.fori_loop(0, _PHB // 2 - 1, body, 0)
        issue(_PHB - 1, a1_v, b1_v, c1_v, sa1, sb1, sc1)
        process(_PHB - 2, a0_v, b0_v, c0_v, sa0, sb0, sc0)
        process(_PHB - 1, a1_v, b1_v, c1_v, sa1, sb1, sc1)
        return carry

    lax.fori_loop(0, nph, phase, 0)

    plsc.subcore_barrier()

    out0 = cid * _NP + row0
    pltpu.sync_copy(s_sh.at[pl.ds(row0, _RPT)], s_out.at[pl.ds(out0, _RPT)])


_sc_call = functools.partial(
    pl.kernel,
    out_type=jax.ShapeDtypeStruct((_NC * _NP, _D), jnp.float32),
    mesh=plsc.VectorSubcoreMesh(core_axis_name="c", subcore_axis_name="s"),
    scratch_types=[
        pltpu.VMEM((_PHB, _EB), jnp.int32),
        pltpu.VMEM((_PHB, _EB), jnp.int32),
        pltpu.VMEM((_EB, _D), jnp.float32),
        pltpu.VMEM((_EB, _D), jnp.float32),
        pltpu.VMEM((_EB, _D), jnp.float32),
        pltpu.VMEM((_EB, _D), jnp.float32),
        pltpu.VMEM((_EB, _D), jnp.float32),
        pltpu.VMEM((_EB, _D), jnp.float32),
        pltpu.VMEM((_D,), jnp.float32),
        pltpu.VMEM_SHARED((_NP, _D), jnp.float32),
        pltpu.SemaphoreType.DMA,
        pltpu.SemaphoreType.DMA,
        pltpu.SemaphoreType.DMA,
        pltpu.SemaphoreType.DMA,
        pltpu.SemaphoreType.DMA,
        pltpu.SemaphoreType.DMA,
    ],
)(_sc_body)


def _final_body(fo_ref, s0_ref, s1_ref, GW2_ref, Wo_ref, out_ref):
    g2o = jnp.dot(GW2_ref[...], Wo_ref[...], preferred_element_type=jnp.float32)
    s = s0_ref[...] + s1_ref[...]
    out_ref[...] = fo_ref[...] + jnp.dot(s, g2o,
                                         preferred_element_type=jnp.float32)


def _final_call(fo, s_p, GW2, Wo):
    rn = 80
    off = _NP // rn
    full = lambda s: pl.BlockSpec(s, lambda i: (0, 0))
    return pl.pallas_call(
        _final_body,
        grid=(_N // rn,),
        in_specs=[
            pl.BlockSpec((rn, 1), lambda i: (i, 0)),
            pl.BlockSpec((rn, _D), lambda i: (i, 0)),
            pl.BlockSpec((rn, _D), lambda i: (i + off, 0)),
            full((_D, _D)), full((_D, 1)),
        ],
        out_specs=pl.BlockSpec((rn, 1), lambda i: (i, 0)),
        out_shape=jax.ShapeDtypeStruct((_N, 1), jnp.float32),
    )(fo, s_p, s_p, GW2, Wo)


def kernel(x, edge_index, theta, w_edge, We, be, FW1, Fb1, FW2, Fb2,
           GW1, Gb1, GW2, Gb2, Wo, bo):
    x2 = x[:, None]
    be2 = be[None, :]
    Fb12 = Fb1[None, :]
    Fb22 = Fb2[None, :]
    Gb12 = Gb1[None, :]
    Gb22 = Gb2[None, :]
    bo2 = bo[None, :]
    npad = _EPAD - _E
    src = jnp.concatenate([edge_index[0], jnp.zeros((npad,), jnp.int32)])
    dst = jnp.concatenate([edge_index[1], jnp.full((npad,), _N, jnp.int32)])
    src = src.reshape(_NS, _TPH, _PHB, _EB)
    dst = dst.reshape(_NS, _TPH, _PHB, _EB)

    a_tab, b_tab, fo, r_col = _node_call(x2, theta, We, be2, FW1, Fb12, FW2,
                                         Fb22, GW1, Gb12, GW2, Gb22, Wo, bo2)
    c_tab = _edge_call(w_edge, GW1[2 * _D:3 * _D, :])

    s_p = _sc_call(src, dst, a_tab, b_tab, c_tab, r_col[:, 0])

    out = _final_call(fo, s_p, GW2, Wo)
    return out[:, 0]


# speedup vs baseline: 1.1490x; 1.1490x over previous
import functools

import jax
import jax.numpy as jnp
from jax import lax
from jax.experimental import pallas as pl
from jax.experimental.pallas import tpu as pltpu
from jax.experimental.pallas import tpu_sc as plsc

_N = 10000
_E = 320000
_D = 128

_NC = 2
_NS = 16
_NW = _NC * _NS
_EB = 40
_TPH = 8
_PH0 = 5
_PHB = 64
_EPP = _PHB * _EB
_EPW = _TPH * _EPP
_EPAD = _NS * _EPW
_NP = 10240
_RPT = _NP // _NS

_RN = 1000
_REB = 4000


def _node_body(x_ref, th_ref, We_ref, be_ref, FW1_ref, Fb1_ref, FW2_ref,
               Fb2_ref, GW1_ref, Gb1_ref, GW2_ref, Gb2_ref, Wo_ref, bo_ref,
               a_ref, b_ref, fo_ref, r_ref):
    h = x_ref[...] * We_ref[...] + be_ref[...]
    z = (jnp.dot(h, FW1_ref[0:_D, :], preferred_element_type=jnp.float32)
         + jnp.dot(th_ref[...], FW1_ref[_D:2 * _D, :],
                   preferred_element_type=jnp.float32)
         + Fb1_ref[...])
    u = jnp.where(z > 0, z, jnp.exp(z) - 1.0)
    w2o = jnp.dot(FW2_ref[...], Wo_ref[...], preferred_element_type=jnp.float32)
    fo_ref[...] = (jnp.dot(u, w2o, preferred_element_type=jnp.float32)
                   + jnp.dot(Fb2_ref[...], Wo_ref[...],
                             preferred_element_type=jnp.float32)
                   + bo_ref[...])
    a_ref[...] = (jnp.dot(h, GW1_ref[0:_D, :],
                          preferred_element_type=jnp.float32) + Gb1_ref[...])
    b_ref[...] = jnp.dot(h, GW1_ref[_D:2 * _D, :],
                         preferred_element_type=jnp.float32)
    g2o = jnp.dot(GW2_ref[...], Wo_ref[...], preferred_element_type=jnp.float32)
    gb2o = jnp.dot(Gb2_ref[...], Wo_ref[...], preferred_element_type=jnp.float32)
    r_ref[...] = (g2o * gb2o) / jnp.sum(g2o * g2o)


def _node_call(x2, theta, We, be2, FW1, Fb12, FW2, Fb22, GW1, Gb12, GW2, Gb22,
               Wo, bo2):
    full = lambda s: pl.BlockSpec(s, lambda i: (0, 0))
    return pl.pallas_call(
        _node_body,
        grid=(_N // _RN,),
        in_specs=[
            pl.BlockSpec((_RN, 1), lambda i: (i, 0)),
            pl.BlockSpec((_RN, _D), lambda i: (i, 0)),
            full((1, _D)), full((1, _D)),
            full((2 * _D, _D)), full((1, _D)),
            full((_D, _D)), full((1, _D)),
            full((3 * _D, _D)), full((1, _D)),
            full((_D, _D)), full((1, _D)),
            full((_D, 1)), full((1, 1)),
        ],
        out_specs=[
            pl.BlockSpec((_RN, _D), lambda i: (i, 0)),
            pl.BlockSpec((_RN, _D), lambda i: (i, 0)),
            pl.BlockSpec((_RN, 1), lambda i: (i, 0)),
            pl.BlockSpec((_D, 1), lambda i: (0, 0)),
        ],
        out_shape=[
            jax.ShapeDtypeStruct((_NP, _D), jnp.float32),
            jax.ShapeDtypeStruct((_NP, _D), jnp.float32),
            jax.ShapeDtypeStruct((_N, 1), jnp.float32),
            jax.ShapeDtypeStruct((_D, 1), jnp.float32),
        ],
    )(x2, theta, We, be2, FW1, Fb12, FW2, Fb22, GW1, Gb12, GW2, Gb22, Wo, bo2)


def _edge_body(w_ref, g_ref, c_ref):
    c_ref[...] = jnp.dot(w_ref[...], g_ref[...],
                         preferred_element_type=jnp.float32)


def _edge_call(w_edge, G1c):
    return pl.pallas_call(
        _edge_body,
        grid=(_E // _REB,),
        in_specs=[
            pl.BlockSpec((_REB, _D), lambda i: (i, 0)),
            pl.BlockSpec((_D, _D), lambda i: (0, 0)),
        ],
        out_specs=pl.BlockSpec((_REB, _D), lambda i: (i, 0)),
        out_shape=jax.ShapeDtypeStruct((_EPAD, _D), jnp.float32),
    )(w_edge, G1c)


def _sc_body(src_hbm, dst_hbm, a_hbm, b_hbm, c_hbm, r_hbm,
             s_out,
             idxs_v, idxd_v, a0_v, b0_v, c0_v, a1_v, b1_v, c1_v, r_v, s_sh,
             sa0, sb0, sc0, sa1, sb1, sc1):
    cid = lax.axis_index("c")
    sid = lax.axis_index("s")

    pltpu.sync_copy(r_hbm, r_v)

    zero16 = jnp.zeros((16,), jnp.float32)
    for q in range(_EB):
        for j in range(_D // 16):
            a0_v[q, pl.ds(j * 16, 16)] = zero16
    row0 = sid * _RPT
    for q in range(_RPT // _EB):
        pltpu.sync_copy(a0_v, s_sh.at[pl.ds(row0 + q * _EB, _EB)])
    plsc.subcore_barrier()

    nph = jnp.where(cid == 0, _PH0, _TPH - _PH0)
    phoff = jnp.where(cid == 0, 0, _PH0)

    def phase(ph, carry):
        phg = phoff + ph
        pltpu.sync_copy(src_hbm.at[sid, phg], idxs_v)
        pltpu.sync_copy(dst_hbm.at[sid, phg], idxd_v)
        base0 = sid * _EPW + phg * _EPP

        def issue(i, av, bv, cv, sa, sb, sc):
            pltpu.async_copy(a_hbm.at[idxd_v.at[i]], av, sa)
            pltpu.async_copy(b_hbm.at[idxs_v.at[i]], bv, sb)
            pltpu.async_copy(c_hbm.at[pl.ds(base0 + i * _EB, _EB)], cv, sc)

        def wait_set(av, bv, cv, sa, sb, sc):
            pltpu.make_async_copy(a_hbm.at[idxd_v.at[0]], av, sa).wait()
            pltpu.make_async_copy(b_hbm.at[idxs_v.at[0]], bv, sb).wait()
            pltpu.make_async_copy(c_hbm.at[pl.ds(base0, _EB)], cv, sc).wait()

        def process(i, av, bv, cv, sa, sb, sc):
            wait_set(av, bv, cv, sa, sb, sc)

            @plsc.parallel_loop(0, _EB, step=1, unroll=4)
            def comp(e):
                for j in range(_D // 16):
                    sl = pl.ds(j * 16, 16)
                    z = av[e, sl] + bv[e, sl] + cv[e, sl]
                    av[e, sl] = jnp.where(z > 0, z, jnp.exp(z) - 1.0) + r_v[sl]

            pltpu.sync_copy(av, s_sh.at[idxd_v.at[i]], add=True)

        issue(0, a0_v, b0_v, c0_v, sa0, sb0, sc0)

        def body(k, inner):
            i0 = 2 * k
            issue(i0 + 1, a1_v, b1_v, c1_v, sa1, sb1, sc1)
            process(i0, a0_v, b0_v, c0_v, sa0, sb0, sc0)
            issue(i0 + 2, a0_v, b0_v, c0_v, sa0, sb0, sc0)
            process(i0 + 1, a1_v, b1_v, c1_v, sa1, sb1, sc1)
            return inner

        lax.fori_loop(0, _PHB // 2 - 1, body, 0)
        issue(_PHB - 1, a1_v, b1_v, c1_v, sa1, sb1, sc1)
        process(_PHB - 2, a0_v, b0_v, c0_v, sa0, sb0, sc0)
        process(_PHB - 1, a1_v, b1_v, c1_v, sa1, sb1, sc1)
        return carry

    lax.fori_loop(0, nph, phase, 0)

    plsc.subcore_barrier()

    out0 = cid * _NP + row0
    pltpu.sync_copy(s_sh.at[pl.ds(row0, _RPT)], s_out.at[pl.ds(out0, _RPT)])


_sc_call = functools.partial(
    pl.kernel,
    out_type=jax.ShapeDtypeStruct((_NC * _NP, _D), jnp.float32),
    mesh=plsc.VectorSubcoreMesh(core_axis_name="c", subcore_axis_name="s"),
    scratch_types=[
        pltpu.VMEM((_PHB, _EB), jnp.int32),
        pltpu.VMEM((_PHB, _EB), jnp.int32),
        pltpu.VMEM((_EB, _D), jnp.float32),
        pltpu.VMEM((_EB, _D), jnp.float32),
        pltpu.VMEM((_EB, _D), jnp.float32),
        pltpu.VMEM((_EB, _D), jnp.float32),
        pltpu.VMEM((_EB, _D), jnp.float32),
        pltpu.VMEM((_EB, _D), jnp.float32),
        pltpu.VMEM((_D,), jnp.float32),
        pltpu.VMEM_SHARED((_NP, _D), jnp.float32),
        pltpu.SemaphoreType.DMA,
        pltpu.SemaphoreType.DMA,
        pltpu.SemaphoreType.DMA,
        pltpu.SemaphoreType.DMA,
        pltpu.SemaphoreType.DMA,
        pltpu.SemaphoreType.DMA,
    ],
)(_sc_body)


def _final_body(fo_ref, s0_ref, s1_ref, GW2_ref, Wo_ref, out_ref):
    g2o = jnp.dot(GW2_ref[...], Wo_ref[...], preferred_element_type=jnp.float32)
    s = s0_ref[...] + s1_ref[...]
    out_ref[...] = fo_ref[...] + jnp.dot(s, g2o,
                                         preferred_element_type=jnp.float32)


def _final_call(fo, s_p, GW2, Wo):
    rn = 80
    off = _NP // rn
    full = lambda s: pl.BlockSpec(s, lambda i: (0, 0))
    return pl.pallas_call(
        _final_body,
        grid=(_N // rn,),
        in_specs=[
            pl.BlockSpec((rn, 1), lambda i: (i, 0)),
            pl.BlockSpec((rn, _D), lambda i: (i, 0)),
            pl.BlockSpec((rn, _D), lambda i: (i + off, 0)),
            full((_D, _D)), full((_D, 1)),
        ],
        out_specs=pl.BlockSpec((rn, 1), lambda i: (i, 0)),
        out_shape=jax.ShapeDtypeStruct((_N, 1), jnp.float32),
    )(fo, s_p, s_p, GW2, Wo)


def kernel(x, edge_index, theta, w_edge, We, be, FW1, Fb1, FW2, Fb2,
           GW1, Gb1, GW2, Gb2, Wo, bo):
    x2 = x[:, None]
    be2 = be[None, :]
    Fb12 = Fb1[None, :]
    Fb22 = Fb2[None, :]
    Gb12 = Gb1[None, :]
    Gb22 = Gb2[None, :]
    bo2 = bo[None, :]
    npad = _EPAD - _E
    src = jnp.concatenate([edge_index[0], jnp.zeros((npad,), jnp.int32)])
    dst = jnp.concatenate([edge_index[1], jnp.full((npad,), _N, jnp.int32)])
    src = src.reshape(_NS, _TPH, _PHB, _EB)
    dst = dst.reshape(_NS, _TPH, _PHB, _EB)

    a_tab, b_tab, fo, r_col = _node_call(x2, theta, We, be2, FW1, Fb12, FW2,
                                         Fb22, GW1, Gb12, GW2, Gb22, Wo, bo2)
    c_tab = _edge_call(w_edge, GW1[2 * _D:3 * _D, :])

    s_p = _sc_call(src, dst, a_tab, b_tab, c_tab, r_col[:, 0])

    out = _final_call(fo, s_p, GW2, Wo)
    return out[:, 0]
